# Initial kernel scaffold; baseline (speedup 1.0000x reference)
import jax
import jax.numpy as jnp
from jax.experimental import pallas as pl


def kernel(h, edge_index, WQ, WK, WV, WO, bO, W1, b1, W2, b2, g1, be1, g2, be2):
    raise NotImplementedError("write your pallas kernel here")


# speedup vs baseline: 5.8726x; 5.8726x over previous
import functools

import jax
import jax.numpy as jnp
import numpy as np
from jax import lax
from jax.experimental import pallas as pl
from jax.experimental.pallas import tpu as pltpu
from jax.experimental.pallas import tpu_sc as plsc

_N = 10000
_E = 160000
_D = 256
_H = 8
_DH = _D // _H
_DHALF = _D // 2
_NC = 2
_NS = 16
_L = 16
_EPW = _E // _NS
_C = 80
_NCHUNK = _EPW // _C
_NPAD = 10240
_RPW = _NPAD // _NS
_RBLK = 128
_INV_SQRT_DH = float(1.0 / np.sqrt(_DH))


def _sc_attention(kh, qh, vh, src, dst):
    mesh = plsc.VectorSubcoreMesh(core_axis_name="c", subcore_axis_name="s")

    @functools.partial(
        pl.kernel,
        out_type=jax.ShapeDtypeStruct((2 * _NPAD, _DHALF), jnp.float32),
        mesh=mesh,
        compiler_params=pltpu.CompilerParams(
            needs_layout_passes=False, use_tc_tiling_on_sc=False),
        scratch_types=[
            pltpu.VMEM((_C,), jnp.int32),
            pltpu.VMEM((_C,), jnp.int32),
            pltpu.VMEM((_C,), jnp.int32),
            pltpu.VMEM((_C, _DHALF), jnp.float32),
            pltpu.VMEM((_C, _DHALF), jnp.float32),
            pltpu.VMEM((_C, _L), jnp.float32),
            pltpu.VMEM_SHARED((_NPAD, _DHALF), jnp.float32),
            pltpu.VMEM_SHARED((_NPAD, _L), jnp.float32),
            pltpu.SemaphoreType.DMA,
        ],
    )
    def attn(kh_hbm, qh_hbm, vh_hbm, src_hbm, dst_hbm, out_hbm,
             srci, dsti, dstio, kbuf, qbuf, sbuf,
             acc, zacc, sem):
        cid = lax.axis_index("c")
        sid = lax.axis_index("s")
        zero = jnp.zeros((_L,), jnp.float32)
        coff = cid * _N

        def zrow(r, carry):
            for j in range(_DHALF // _L):
                kbuf[r, pl.ds(j * _L, _L)] = zero
            sbuf[r, :] = zero
            return carry

        lax.fori_loop(0, _C, zrow, 0)
        for i in range(_RPW // _C):
            r0 = sid * _RPW + i * _C
            pltpu.sync_copy(kbuf, acc.at[pl.ds(r0, _C)])
            pltpu.sync_copy(sbuf, zacc.at[pl.ds(r0, _C)])
        plsc.subcore_barrier()

        def chunk(ck, carry):
            base = sid * _EPW + ck * _C
            pltpu.sync_copy(src_hbm.at[pl.ds(base, _C)], srci)
            pltpu.sync_copy(dst_hbm.at[pl.ds(base, _C)], dsti)
            for j in range(_C // _L):
                sl = pl.ds(j * _L, _L)
                srci[sl] = srci[sl] + coff
                dstio[sl] = dsti[sl] + coff
            pltpu.async_copy(kh_hbm.at[srci], kbuf, sem).wait()
            pltpu.async_copy(qh_hbm.at[dstio], qbuf, sem).wait()

            lanes = lax.iota(jnp.int32, _L)

            def score_body(g, c2):
                rows = g * _L + lanes
                for hh in range(4):
                    a16 = jnp.zeros((_L,), jnp.float32)
                    for d in range(_DH):
                        col = jnp.full((_L,), hh * _DH + d, jnp.int32)
                        kk = plsc.load_gather(kbuf, [rows, col])
                        qq = plsc.load_gather(qbuf, [rows, col])
                        a16 = a16 + kk * qq
                    s16 = jnp.exp(jnp.clip(a16 * _INV_SQRT_DH, -5.0, 5.0))
                    plsc.store_scatter(
                        sbuf, [rows, jnp.full((_L,), hh, jnp.int32)], s16)
                return c2

            lax.fori_loop(0, _C // _L, score_body, 0)

            pltpu.async_copy(vh_hbm.at[srci], qbuf, sem).wait()

            def scale_body(g, c2):
                rows = g * _L + lanes
                for hh in range(4):
                    s16 = plsc.load_gather(
                        sbuf, [rows, jnp.full((_L,), hh, jnp.int32)])
                    for d in range(_DH):
                        col = jnp.full((_L,), hh * _DH + d, jnp.int32)
                        vv = plsc.load_gather(qbuf, [rows, col])
                        plsc.store_scatter(qbuf, [rows, col], vv * s16)
                return c2

            lax.fori_loop(0, _C // _L, scale_body, 0)

            pltpu.sync_copy(qbuf, acc.at[dsti], add=True)
            pltpu.sync_copy(sbuf, zacc.at[dsti], add=True)
            return carry

        lax.fori_loop(0, _NCHUNK, chunk, 0)
        plsc.subcore_barrier()

        for i in range(_RPW // _C):
            r0 = sid * _RPW + i * _C
            pltpu.sync_copy(acc.at[pl.ds(r0, _C)], kbuf)
            pltpu.sync_copy(zacc.at[pl.ds(r0, _C)], sbuf)

            def div_body(r, c2):
                zrow = 1.0 / sbuf[r, :]
                for hh in range(4):
                    zr = zrow[hh]
                    for half in range(2):
                        sl = pl.ds(hh * _DH + half * _L, _L)
                        kbuf[r, sl] = kbuf[r, sl] * zr
                return c2

            lax.fori_loop(0, _C, div_body, 0)
            pltpu.sync_copy(kbuf, out_hbm.at[pl.ds(cid * _NPAD + r0, _C)])

    return attn(kh, qh, vh, src, dst)


_BM = 1000
_GRID = _N // _BM


def _tc_qkv(hm, w):
    def body(x_ref, w_ref, o_ref):
        o_ref[...] = jnp.dot(x_ref[...], w_ref[...],
                             preferred_element_type=jnp.float32)

    return pl.pallas_call(
        body,
        grid=(_GRID,),
        in_specs=[pl.BlockSpec((_BM, _D), lambda i: (i, 0)),
                  pl.BlockSpec((_D, 3 * _D), lambda i: (0, 0))],
        out_specs=pl.BlockSpec((_BM, 3 * _D), lambda i: (i, 0)),
        out_shape=jax.ShapeDtypeStruct((_N, 3 * _D), jnp.float32),
    )(hm, w)


def _tc_attn_out(attn, wo, bo, hm):
    def body(a_ref, w_ref, b_ref, h_ref, u_ref, s1_ref, s2_ref):
        u = jnp.dot(a_ref[...], w_ref[...],
                    preferred_element_type=jnp.float32)
        u = u + b_ref[...] + h_ref[...]
        u_ref[...] = u
        s1_ref[...] = jnp.broadcast_to(jnp.sum(u, axis=0, keepdims=True),
                                       (1, 8, _D))
        s2_ref[...] = jnp.broadcast_to(jnp.sum(u * u, axis=0, keepdims=True),
                                       (1, 8, _D))

    return pl.pallas_call(
        body,
        grid=(_GRID,),
        in_specs=[pl.BlockSpec((_BM, _D), lambda i: (i, 0)),
                  pl.BlockSpec((_D, _D), lambda i: (0, 0)),
                  pl.BlockSpec((1, _D), lambda i: (0, 0)),
                  pl.BlockSpec((_BM, _D), lambda i: (i, 0))],
        out_specs=[pl.BlockSpec((_BM, _D), lambda i: (i, 0)),
                   pl.BlockSpec((1, 8, _D), lambda i: (i, 0, 0)),
                   pl.BlockSpec((1, 8, _D), lambda i: (i, 0, 0))],
        out_shape=[jax.ShapeDtypeStruct((_N, _D), jnp.float32),
                   jax.ShapeDtypeStruct((_GRID, 8, _D), jnp.float32),
                   jax.ShapeDtypeStruct((_GRID, 8, _D), jnp.float32)],
    )(attn, wo, bo, hm)


def _tc_ffn(u, sc1, sh1, w1, b1, w2, b2):
    def body(u_ref, sc_ref, sh_ref, w1_ref, b1_ref, w2_ref, b2_ref,
             v_ref, s1_ref, s2_ref):
        y = u_ref[...] * sc_ref[...] + sh_ref[...]
        t = jnp.dot(y, w1_ref[...], preferred_element_type=jnp.float32)
        t = jnp.maximum(t + b1_ref[...], 0.0)
        f = jnp.dot(t, w2_ref[...], preferred_element_type=jnp.float32)
        v = y + f + b2_ref[...]
        v_ref[...] = v
        s1_ref[...] = jnp.broadcast_to(jnp.sum(v, axis=0, keepdims=True),
                                       (1, 8, _D))
        s2_ref[...] = jnp.broadcast_to(jnp.sum(v * v, axis=0, keepdims=True),
                                       (1, 8, _D))

    return pl.pallas_call(
        body,
        grid=(_GRID,),
        in_specs=[pl.BlockSpec((_BM, _D), lambda i: (i, 0)),
                  pl.BlockSpec((1, _D), lambda i: (0, 0)),
                  pl.BlockSpec((1, _D), lambda i: (0, 0)),
                  pl.BlockSpec((_D, 2 * _D), lambda i: (0, 0)),
                  pl.BlockSpec((1, 2 * _D), lambda i: (0, 0)),
                  pl.BlockSpec((2 * _D, _D), lambda i: (0, 0)),
                  pl.BlockSpec((1, _D), lambda i: (0, 0))],
        out_specs=[pl.BlockSpec((_BM, _D), lambda i: (i, 0)),
                   pl.BlockSpec((1, 8, _D), lambda i: (i, 0, 0)),
                   pl.BlockSpec((1, 8, _D), lambda i: (i, 0, 0))],
        out_shape=[jax.ShapeDtypeStruct((_N, _D), jnp.float32),
                   jax.ShapeDtypeStruct((_GRID, 8, _D), jnp.float32),
                   jax.ShapeDtypeStruct((_GRID, 8, _D), jnp.float32)],
    )(u, sc1, sh1, w1, b1, w2, b2)


def _tc_bn(v, sc2, sh2):
    def body(v_ref, sc_ref, sh_ref, o_ref):
        o_ref[...] = v_ref[...] * sc_ref[...] + sh_ref[...]

    return pl.pallas_call(
        body,
        grid=(_GRID,),
        in_specs=[pl.BlockSpec((_BM, _D), lambda i: (i, 0)),
                  pl.BlockSpec((1, _D), lambda i: (0, 0)),
                  pl.BlockSpec((1, _D), lambda i: (0, 0))],
        out_specs=pl.BlockSpec((_BM, _D), lambda i: (i, 0)),
        out_shape=jax.ShapeDtypeStruct((_N, _D), jnp.float32),
    )(v, sc2, sh2)


def kernel(h, edge_index, WQ, WK, WV, WO, bO, W1, b1, W2, b2, g1, be1,
           g2, be2):
    src = edge_index[0].astype(jnp.int32)
    dst = edge_index[1].astype(jnp.int32)

    wqkv = jnp.concatenate([WQ, WK, WV], axis=1)
    qkv = _tc_qkv(h, wqkv)
    q = qkv[:, :_D]
    k = qkv[:, _D:2 * _D]
    v = qkv[:, 2 * _D:]

    kh = jnp.concatenate([k[:, :_DHALF], k[:, _DHALF:]], axis=0)
    qh = jnp.concatenate([q[:, :_DHALF], q[:, _DHALF:]], axis=0)
    vh = jnp.concatenate([v[:, :_DHALF], v[:, _DHALF:]], axis=0)

    ho = _sc_attention(kh, qh, vh, src, dst)
    attn = jnp.concatenate([ho[:_N], ho[_NPAD:_NPAD + _N]], axis=1)

    u, s1, s2 = _tc_attn_out(attn, WO, bO.reshape(1, _D), h)
    mean1 = jnp.sum(s1[:, 0, :], axis=0) / _N
    var1 = jnp.sum(s2[:, 0, :], axis=0) / _N - mean1 * mean1
    g1i = g1 / jnp.sqrt(var1 + 1e-5)
    sc1 = g1i.reshape(1, _D)
    sh1 = (be1 - mean1 * g1i).reshape(1, _D)

    vv, t1, t2 = _tc_ffn(u, sc1, sh1, W1, b1.reshape(1, 2 * _D), W2,
                         b2.reshape(1, _D))
    mean2 = jnp.sum(t1[:, 0, :], axis=0) / _N
    var2 = jnp.sum(t2[:, 0, :], axis=0) / _N - mean2 * mean2
    g2i = g2 / jnp.sqrt(var2 + 1e-5)
    sc2 = g2i.reshape(1, _D)
    sh2 = (be2 - mean2 * g2i).reshape(1, _D)

    return _tc_bn(vv, sc2, sh2)


# speedup vs baseline: 6.2051x; 1.0566x over previous
import functools

import jax
import jax.numpy as jnp
import numpy as np
from jax import lax
from jax.experimental import pallas as pl
from jax.experimental.pallas import tpu as pltpu
from jax.experimental.pallas import tpu_sc as plsc

_N = 10000
_E = 160000
_D = 256
_H = 8
_DH = _D // _H
_DHALF = _D // 2
_NC = 2
_NS = 16
_L = 16
_EPW = _E // _NS
_C = 80
_NCHUNK = _EPW // _C
_NPAD = 10240
_RPW = _NPAD // _NS
_RBLK = 128
_INV_SQRT_DH = float(1.0 / np.sqrt(_DH))


def _sc_attention(tbl, packed):
    mesh = plsc.VectorSubcoreMesh(core_axis_name="c", subcore_axis_name="s")

    @functools.partial(
        pl.kernel,
        out_type=jax.ShapeDtypeStruct((2 * _NPAD, _DHALF), jnp.float32),
        mesh=mesh,
        compiler_params=pltpu.CompilerParams(
            needs_layout_passes=False, use_tc_tiling_on_sc=False),
        scratch_types=[
            pltpu.VMEM((2 * _C,), jnp.int32),
            pltpu.VMEM((2 * _C,), jnp.int32),
            pltpu.VMEM((_C,), jnp.int32),
            pltpu.VMEM((_C,), jnp.int32),
            pltpu.VMEM((2 * _C, _DHALF), jnp.float32),
            pltpu.VMEM((_C, _L), jnp.float32),
            pltpu.VMEM((_C, _L), jnp.float32),
            pltpu.VMEM_SHARED((_NPAD, _DHALF), jnp.float32),
            pltpu.VMEM_SHARED((_NPAD, _L), jnp.float32),
            pltpu.SemaphoreType.DMA,
            pltpu.SemaphoreType.DMA,
            pltpu.SemaphoreType.DMA,
            pltpu.SemaphoreType.DMA,
            pltpu.SemaphoreType.DMA,
        ],
    )
    def attn(tbl_hbm, packed_hbm, out_hbm,
             idx0, idx1, dst0, dst1, kqbuf, sbuf0, sbuf1,
             acc, zacc, semg, sa0, sa1, sz0, sz1):
        cid = lax.axis_index("c")
        sid = lax.axis_index("s")
        zero = jnp.zeros((_L,), jnp.float32)
        coff = cid * _N
        lanes = lax.iota(jnp.int32, _L)

        def zrow(r, carry):
            for j in range(_DHALF // _L):
                kqbuf[r, pl.ds(j * _L, _L)] = zero
            sbuf0[r, :] = zero
            return carry

        lax.fori_loop(0, _C, zrow, 0)
        for i in range(_RPW // _C):
            r0 = sid * _RPW + i * _C
            pltpu.sync_copy(kqbuf.at[pl.ds(0, _C)], acc.at[pl.ds(r0, _C)])
            pltpu.sync_copy(sbuf0, zacc.at[pl.ds(r0, _C)])
        plsc.subcore_barrier()

        def chunk_steps(ck, idxb, dstb, sbufb, sema, semz, wait_prev):
            base = (sid * _NCHUNK + ck) * (2 * _C)
            pltpu.sync_copy(packed_hbm.at[pl.ds(base, 2 * _C)], idxb)
            for j in range(_C // _L):
                sl = pl.ds(j * _L, _L)
                idxb[sl] = idxb[sl] + coff
            for j in range(_C // _L):
                sl = pl.ds(_C + j * _L, _L)
                d = idxb[sl]
                dstb[pl.ds(j * _L, _L)] = d
                idxb[sl] = d + (2 * _N) + coff
            if wait_prev is not None:
                qdst, qsb, qsa, qsz = wait_prev
                pltpu.make_async_copy(
                    kqbuf.at[pl.ds(0, _C)], acc.at[qdst], qsa).wait()
                pltpu.make_async_copy(qsb, zacc.at[qdst], qsz).wait()
            pltpu.async_copy(tbl_hbm.at[idxb], kqbuf, semg).wait()

            def score_body(g, c2):
                rows = g * _L + lanes
                for hh in range(4):
                    a16 = jnp.zeros((_L,), jnp.float32)
                    for d in range(_DH):
                        col = jnp.full((_L,), hh * _DH + d, jnp.int32)
                        kk = plsc.load_gather(kqbuf, [rows, col])
                        qq = plsc.load_gather(kqbuf, [rows + _C, col])
                        a16 = a16 + kk * qq
                    s16 = jnp.exp(jnp.clip(a16 * _INV_SQRT_DH, -5.0, 5.0))
                    plsc.store_scatter(
                        sbufb, [rows, jnp.full((_L,), hh, jnp.int32)], s16)
                return c2

            lax.fori_loop(0, _C // _L, score_body, 0)

            for j in range(_C // _L):
                sl = pl.ds(j * _L, _L)
                idxb[sl] = idxb[sl] + 4 * _N
            pltpu.async_copy(
                tbl_hbm.at[idxb.at[pl.ds(0, _C)]],
                kqbuf.at[pl.ds(0, _C)], semg).wait()

            def scale_body(g, c2):
                rows = g * _L + lanes
                for hh in range(4):
                    s16 = plsc.load_gather(
                        sbufb, [rows, jnp.full((_L,), hh, jnp.int32)])
                    for d in range(_DH):
                        col = jnp.full((_L,), hh * _DH + d, jnp.int32)
                        vv = plsc.load_gather(kqbuf, [rows, col])
                        plsc.store_scatter(kqbuf, [rows, col], vv * s16)
                return c2

            lax.fori_loop(0, _C // _L, scale_body, 0)

            pltpu.async_copy(
                kqbuf.at[pl.ds(0, _C)], acc.at[dstb], sema, add=True)
            pltpu.async_copy(sbufb, zacc.at[dstb], semz, add=True)

        chunk_steps(0, idx0, dst0, sbuf0, sa0, sz0, None)

        def pair(j, carry):
            chunk_steps(1 + 2 * j, idx1, dst1, sbuf1, sa1, sz1,
                        (dst0, sbuf0, sa0, sz0))
            chunk_steps(2 + 2 * j, idx0, dst0, sbuf0, sa0, sz0,
                        (dst1, sbuf1, sa1, sz1))
            return carry

        lax.fori_loop(0, (_NCHUNK - 1) // 2, pair, 0)

        pltpu.make_async_copy(
            kqbuf.at[pl.ds(0, _C)], acc.at[dst0], sa0).wait()
        pltpu.make_async_copy(sbuf0, zacc.at[dst0], sz0).wait()
        plsc.subcore_barrier()

        for i in range(_RPW // _C):
            r0 = sid * _RPW + i * _C
            pltpu.sync_copy(acc.at[pl.ds(r0, _C)], kqbuf.at[pl.ds(0, _C)])
            pltpu.sync_copy(zacc.at[pl.ds(r0, _C)], sbuf0)

            def div_body(r, c2):
                zrow = 1.0 / sbuf0[r, :]
                for hh in range(4):
                    zr = zrow[hh]
                    for half in range(2):
                        sl = pl.ds(hh * _DH + half * _L, _L)
                        kqbuf[r, sl] = kqbuf[r, sl] * zr
                return c2

            lax.fori_loop(0, _C, div_body, 0)
            pltpu.sync_copy(kqbuf.at[pl.ds(0, _C)],
                            out_hbm.at[pl.ds(cid * _NPAD + r0, _C)])

    return attn(tbl, packed)


_BM = 1000
_GRID = _N // _BM


def _tc_qkv(hm, w):
    def body(x_ref, w_ref, o_ref):
        o_ref[...] = jnp.dot(x_ref[...], w_ref[...],
                             preferred_element_type=jnp.float32)

    return pl.pallas_call(
        body,
        grid=(_GRID,),
        in_specs=[pl.BlockSpec((_BM, _D), lambda i: (i, 0)),
                  pl.BlockSpec((_D, 3 * _D), lambda i: (0, 0))],
        out_specs=pl.BlockSpec((_BM, 3 * _D), lambda i: (i, 0)),
        out_shape=jax.ShapeDtypeStruct((_N, 3 * _D), jnp.float32),
    )(hm, w)


def _tc_attn_out(attn, wo, bo, hm):
    def body(a_ref, w_ref, b_ref, h_ref, u_ref, s1_ref, s2_ref):
        u = jnp.dot(a_ref[...], w_ref[...],
                    preferred_element_type=jnp.float32)
        u = u + b_ref[...] + h_ref[...]
        u_ref[...] = u
        s1_ref[...] = jnp.broadcast_to(jnp.sum(u, axis=0, keepdims=True),
                                       (1, 8, _D))
        s2_ref[...] = jnp.broadcast_to(jnp.sum(u * u, axis=0, keepdims=True),
                                       (1, 8, _D))

    return pl.pallas_call(
        body,
        grid=(_GRID,),
        in_specs=[pl.BlockSpec((_BM, _D), lambda i: (i, 0)),
                  pl.BlockSpec((_D, _D), lambda i: (0, 0)),
                  pl.BlockSpec((1, _D), lambda i: (0, 0)),
                  pl.BlockSpec((_BM, _D), lambda i: (i, 0))],
        out_specs=[pl.BlockSpec((_BM, _D), lambda i: (i, 0)),
                   pl.BlockSpec((1, 8, _D), lambda i: (i, 0, 0)),
                   pl.BlockSpec((1, 8, _D), lambda i: (i, 0, 0))],
        out_shape=[jax.ShapeDtypeStruct((_N, _D), jnp.float32),
                   jax.ShapeDtypeStruct((_GRID, 8, _D), jnp.float32),
                   jax.ShapeDtypeStruct((_GRID, 8, _D), jnp.float32)],
    )(attn, wo, bo, hm)


def _tc_ffn(u, sc1, sh1, w1, b1, w2, b2):
    def body(u_ref, sc_ref, sh_ref, w1_ref, b1_ref, w2_ref, b2_ref,
             v_ref, s1_ref, s2_ref):
        y = u_ref[...] * sc_ref[...] + sh_ref[...]
        t = jnp.dot(y, w1_ref[...], preferred_element_type=jnp.float32)
        t = jnp.maximum(t + b1_ref[...], 0.0)
        f = jnp.dot(t, w2_ref[...], preferred_element_type=jnp.float32)
        v = y + f + b2_ref[...]
        v_ref[...] = v
        s1_ref[...] = jnp.broadcast_to(jnp.sum(v, axis=0, keepdims=True),
                                       (1, 8, _D))
        s2_ref[...] = jnp.broadcast_to(jnp.sum(v * v, axis=0, keepdims=True),
                                       (1, 8, _D))

    return pl.pallas_call(
        body,
        grid=(_GRID,),
        in_specs=[pl.BlockSpec((_BM, _D), lambda i: (i, 0)),
                  pl.BlockSpec((1, _D), lambda i: (0, 0)),
                  pl.BlockSpec((1, _D), lambda i: (0, 0)),
                  pl.BlockSpec((_D, 2 * _D), lambda i: (0, 0)),
                  pl.BlockSpec((1, 2 * _D), lambda i: (0, 0)),
                  pl.BlockSpec((2 * _D, _D), lambda i: (0, 0)),
                  pl.BlockSpec((1, _D), lambda i: (0, 0))],
        out_specs=[pl.BlockSpec((_BM, _D), lambda i: (i, 0)),
                   pl.BlockSpec((1, 8, _D), lambda i: (i, 0, 0)),
                   pl.BlockSpec((1, 8, _D), lambda i: (i, 0, 0))],
        out_shape=[jax.ShapeDtypeStruct((_N, _D), jnp.float32),
                   jax.ShapeDtypeStruct((_GRID, 8, _D), jnp.float32),
                   jax.ShapeDtypeStruct((_GRID, 8, _D), jnp.float32)],
    )(u, sc1, sh1, w1, b1, w2, b2)


def _tc_bn(v, sc2, sh2):
    def body(v_ref, sc_ref, sh_ref, o_ref):
        o_ref[...] = v_ref[...] * sc_ref[...] + sh_ref[...]

    return pl.pallas_call(
        body,
        grid=(_GRID,),
        in_specs=[pl.BlockSpec((_BM, _D), lambda i: (i, 0)),
                  pl.BlockSpec((1, _D), lambda i: (0, 0)),
                  pl.BlockSpec((1, _D), lambda i: (0, 0))],
        out_specs=pl.BlockSpec((_BM, _D), lambda i: (i, 0)),
        out_shape=jax.ShapeDtypeStruct((_N, _D), jnp.float32),
    )(v, sc2, sh2)


def kernel(h, edge_index, WQ, WK, WV, WO, bO, W1, b1, W2, b2, g1, be1,
           g2, be2):
    src = edge_index[0].astype(jnp.int32)
    dst = edge_index[1].astype(jnp.int32)

    wqkv = jnp.concatenate([WQ, WK, WV], axis=1)
    qkv = _tc_qkv(h, wqkv)
    q = qkv[:, :_D]
    k = qkv[:, _D:2 * _D]
    v = qkv[:, 2 * _D:]

    tbl = jnp.concatenate(
        [k[:, :_DHALF], k[:, _DHALF:],
         q[:, :_DHALF], q[:, _DHALF:],
         v[:, :_DHALF], v[:, _DHALF:]], axis=0)

    src2 = src.reshape(_NS, _NCHUNK, _C)
    dst2 = dst.reshape(_NS, _NCHUNK, _C)
    packed = jnp.stack([src2, dst2], axis=2).reshape(-1)

    ho = _sc_attention(tbl, packed)
    attn = jnp.concatenate([ho[:_N], ho[_NPAD:_NPAD + _N]], axis=1)

    u, s1, s2 = _tc_attn_out(attn, WO, bO.reshape(1, _D), h)
    mean1 = jnp.sum(s1[:, 0, :], axis=0) / _N
    var1 = jnp.sum(s2[:, 0, :], axis=0) / _N - mean1 * mean1
    g1i = g1 / jnp.sqrt(var1 + 1e-5)
    sc1 = g1i.reshape(1, _D)
    sh1 = (be1 - mean1 * g1i).reshape(1, _D)

    vv, t1, t2 = _tc_ffn(u, sc1, sh1, W1, b1.reshape(1, 2 * _D), W2,
                         b2.reshape(1, _D))
    mean2 = jnp.sum(t1[:, 0, :], axis=0) / _N
    var2 = jnp.sum(t2[:, 0, :], axis=0) / _N - mean2 * mean2
    g2i = g2 / jnp.sqrt(var2 + 1e-5)
    sc2 = g2i.reshape(1, _D)
    sh2 = (be2 - mean2 * g2i).reshape(1, _D)

    return _tc_bn(vv, sc2, sh2)


# speedup vs baseline: 21.9535x; 3.5380x over previous
import functools

import jax
import jax.numpy as jnp
import numpy as np
from jax import lax
from jax.experimental import pallas as pl
from jax.experimental.pallas import tpu as pltpu
from jax.experimental.pallas import tpu_sc as plsc

_N = 10000
_E = 160000
_D = 256
_H = 8
_DH = _D // _H
_DHALF = _D // 2
_NC = 2
_NS = 16
_L = 16
_EPW = _E // _NS
_C = 80
_NCHUNK = _EPW // _C
_NPAD = 10240
_RPW = _NPAD // _NS
_RBLK = 128
_INV_SQRT_DH = float(1.0 / np.sqrt(_DH))


def _sc_attention(tbl, packed):
    mesh = plsc.VectorSubcoreMesh(core_axis_name="c", subcore_axis_name="s")

    @functools.partial(
        pl.kernel,
        out_type=jax.ShapeDtypeStruct((2 * _NPAD, _DHALF), jnp.float32),
        mesh=mesh,
        compiler_params=pltpu.CompilerParams(
            needs_layout_passes=False, use_tc_tiling_on_sc=False),
        scratch_types=[
            pltpu.VMEM((2 * _C,), jnp.int32),
            pltpu.VMEM((2 * _C,), jnp.int32),
            pltpu.VMEM((_C,), jnp.int32),
            pltpu.VMEM((_C,), jnp.int32),
            pltpu.VMEM((2 * _C, _DHALF), jnp.float32),
            pltpu.VMEM((_C, _L), jnp.float32),
            pltpu.VMEM((_C, _L), jnp.float32),
            pltpu.VMEM_SHARED((_NPAD, _DHALF), jnp.float32),
            pltpu.VMEM_SHARED((_NPAD, _L), jnp.float32),
            pltpu.SemaphoreType.DMA,
            pltpu.SemaphoreType.DMA,
            pltpu.SemaphoreType.DMA,
            pltpu.SemaphoreType.DMA,
            pltpu.SemaphoreType.DMA,
        ],
    )
    def attn(tbl_hbm, packed_hbm, out_hbm,
             idx0, idx1, dst0, dst1, kqbuf, sbuf0, sbuf1,
             acc, zacc, semg, sa0, sa1, sz0, sz1):
        cid = lax.axis_index("c")
        sid = lax.axis_index("s")
        zero = jnp.zeros((_L,), jnp.float32)
        coff = cid * _N
        lanes = lax.iota(jnp.int32, _L)

        def zrow(r, carry):
            for j in range(_DHALF // _L):
                kqbuf[r, pl.ds(j * _L, _L)] = zero
            sbuf0[r, :] = zero
            return carry

        lax.fori_loop(0, _C, zrow, 0)
        for i in range(_RPW // _C):
            r0 = sid * _RPW + i * _C
            pltpu.sync_copy(kqbuf.at[pl.ds(0, _C)], acc.at[pl.ds(r0, _C)])
            pltpu.sync_copy(sbuf0, zacc.at[pl.ds(r0, _C)])
        plsc.subcore_barrier()

        def chunk_steps(ck, idxb, dstb, sbufb, sema, semz, wait_prev):
            base = (sid * _NCHUNK + ck) * (2 * _C)
            pltpu.sync_copy(packed_hbm.at[pl.ds(base, 2 * _C)], idxb)
            for j in range(_C // _L):
                sl = pl.ds(j * _L, _L)
                idxb[sl] = idxb[sl] + coff
            for j in range(_C // _L):
                sl = pl.ds(_C + j * _L, _L)
                d = idxb[sl]
                dstb[pl.ds(j * _L, _L)] = d
                idxb[sl] = d + (2 * _N) + coff
            if wait_prev is not None:
                qdst, qsb, qsa, qsz = wait_prev
                pltpu.make_async_copy(
                    kqbuf.at[pl.ds(0, _C)], acc.at[qdst], qsa).wait()
                pltpu.make_async_copy(qsb, zacc.at[qdst], qsz).wait()
            pltpu.async_copy(tbl_hbm.at[idxb], kqbuf, semg).wait()

            def score_body(e, c2):
                row = jnp.zeros((_L,), jnp.float32)
                for hh in range(4):
                    sl0 = pl.ds(hh * _DH, _L)
                    sl1 = pl.ds(hh * _DH + _L, _L)
                    t = (kqbuf[e, sl0] * kqbuf[e + _C, sl0]
                         + kqbuf[e, sl1] * kqbuf[e + _C, sl1])
                    row = jnp.where(lanes == hh, jnp.sum(t), row)
                sbufb[e, :] = jnp.exp(jnp.clip(row * _INV_SQRT_DH, -5.0, 5.0))
                return c2

            lax.fori_loop(0, _C, score_body, 0)

            for j in range(_C // _L):
                sl = pl.ds(j * _L, _L)
                idxb[sl] = idxb[sl] + 4 * _N
            pltpu.async_copy(
                tbl_hbm.at[idxb.at[pl.ds(0, _C)]],
                kqbuf.at[pl.ds(0, _C)], semg).wait()

            def scale_body(e, c2):
                srow = sbufb[e, :]
                for hh in range(4):
                    s = srow[hh]
                    for half in range(2):
                        sl = pl.ds(hh * _DH + half * _L, _L)
                        kqbuf[e, sl] = kqbuf[e, sl] * s
                return c2

            lax.fori_loop(0, _C, scale_body, 0)

            pltpu.async_copy(
                kqbuf.at[pl.ds(0, _C)], acc.at[dstb], sema, add=True)
            pltpu.async_copy(sbufb, zacc.at[dstb], semz, add=True)

        chunk_steps(0, idx0, dst0, sbuf0, sa0, sz0, None)

        def pair(j, carry):
            chunk_steps(1 + 2 * j, idx1, dst1, sbuf1, sa1, sz1,
                        (dst0, sbuf0, sa0, sz0))
            chunk_steps(2 + 2 * j, idx0, dst0, sbuf0, sa0, sz0,
                        (dst1, sbuf1, sa1, sz1))
            return carry

        lax.fori_loop(0, (_NCHUNK - 1) // 2, pair, 0)

        pltpu.make_async_copy(
            kqbuf.at[pl.ds(0, _C)], acc.at[dst0], sa0).wait()
        pltpu.make_async_copy(sbuf0, zacc.at[dst0], sz0).wait()
        plsc.subcore_barrier()

        for i in range(_RPW // _C):
            r0 = sid * _RPW + i * _C
            pltpu.sync_copy(acc.at[pl.ds(r0, _C)], kqbuf.at[pl.ds(0, _C)])
            pltpu.sync_copy(zacc.at[pl.ds(r0, _C)], sbuf0)

            def div_body(r, c2):
                zrow = 1.0 / sbuf0[r, :]
                for hh in range(4):
                    zr = zrow[hh]
                    for half in range(2):
                        sl = pl.ds(hh * _DH + half * _L, _L)
                        kqbuf[r, sl] = kqbuf[r, sl] * zr
                return c2

            lax.fori_loop(0, _C, div_body, 0)
            pltpu.sync_copy(kqbuf.at[pl.ds(0, _C)],
                            out_hbm.at[pl.ds(cid * _NPAD + r0, _C)])

    return attn(tbl, packed)


_BM = 1000
_GRID = _N // _BM


def _tc_qkv(hm, w):
    def body(x_ref, w_ref, o_ref):
        o_ref[...] = jnp.dot(x_ref[...], w_ref[...],
                             preferred_element_type=jnp.float32)

    return pl.pallas_call(
        body,
        grid=(_GRID,),
        in_specs=[pl.BlockSpec((_BM, _D), lambda i: (i, 0)),
                  pl.BlockSpec((_D, 3 * _D), lambda i: (0, 0))],
        out_specs=pl.BlockSpec((_BM, 3 * _D), lambda i: (i, 0)),
        out_shape=jax.ShapeDtypeStruct((_N, 3 * _D), jnp.float32),
    )(hm, w)


def _tc_attn_out(attn, wo, bo, hm):
    def body(a_ref, w_ref, b_ref, h_ref, u_ref, s1_ref, s2_ref):
        u = jnp.dot(a_ref[...], w_ref[...],
                    preferred_element_type=jnp.float32)
        u = u + b_ref[...] + h_ref[...]
        u_ref[...] = u
        s1_ref[...] = jnp.broadcast_to(jnp.sum(u, axis=0, keepdims=True),
                                       (1, 8, _D))
        s2_ref[...] = jnp.broadcast_to(jnp.sum(u * u, axis=0, keepdims=True),
                                       (1, 8, _D))

    return pl.pallas_call(
        body,
        grid=(_GRID,),
        in_specs=[pl.BlockSpec((_BM, _D), lambda i: (i, 0)),
                  pl.BlockSpec((_D, _D), lambda i: (0, 0)),
                  pl.BlockSpec((1, _D), lambda i: (0, 0)),
                  pl.BlockSpec((_BM, _D), lambda i: (i, 0))],
        out_specs=[pl.BlockSpec((_BM, _D), lambda i: (i, 0)),
                   pl.BlockSpec((1, 8, _D), lambda i: (i, 0, 0)),
                   pl.BlockSpec((1, 8, _D), lambda i: (i, 0, 0))],
        out_shape=[jax.ShapeDtypeStruct((_N, _D), jnp.float32),
                   jax.ShapeDtypeStruct((_GRID, 8, _D), jnp.float32),
                   jax.ShapeDtypeStruct((_GRID, 8, _D), jnp.float32)],
    )(attn, wo, bo, hm)


def _tc_ffn(u, sc1, sh1, w1, b1, w2, b2):
    def body(u_ref, sc_ref, sh_ref, w1_ref, b1_ref, w2_ref, b2_ref,
             v_ref, s1_ref, s2_ref):
        y = u_ref[...] * sc_ref[...] + sh_ref[...]
        t = jnp.dot(y, w1_ref[...], preferred_element_type=jnp.float32)
        t = jnp.maximum(t + b1_ref[...], 0.0)
        f = jnp.dot(t, w2_ref[...], preferred_element_type=jnp.float32)
        v = y + f + b2_ref[...]
        v_ref[...] = v
        s1_ref[...] = jnp.broadcast_to(jnp.sum(v, axis=0, keepdims=True),
                                       (1, 8, _D))
        s2_ref[...] = jnp.broadcast_to(jnp.sum(v * v, axis=0, keepdims=True),
                                       (1, 8, _D))

    return pl.pallas_call(
        body,
        grid=(_GRID,),
        in_specs=[pl.BlockSpec((_BM, _D), lambda i: (i, 0)),
                  pl.BlockSpec((1, _D), lambda i: (0, 0)),
                  pl.BlockSpec((1, _D), lambda i: (0, 0)),
                  pl.BlockSpec((_D, 2 * _D), lambda i: (0, 0)),
                  pl.BlockSpec((1, 2 * _D), lambda i: (0, 0)),
                  pl.BlockSpec((2 * _D, _D), lambda i: (0, 0)),
                  pl.BlockSpec((1, _D), lambda i: (0, 0))],
        out_specs=[pl.BlockSpec((_BM, _D), lambda i: (i, 0)),
                   pl.BlockSpec((1, 8, _D), lambda i: (i, 0, 0)),
                   pl.BlockSpec((1, 8, _D), lambda i: (i, 0, 0))],
        out_shape=[jax.ShapeDtypeStruct((_N, _D), jnp.float32),
                   jax.ShapeDtypeStruct((_GRID, 8, _D), jnp.float32),
                   jax.ShapeDtypeStruct((_GRID, 8, _D), jnp.float32)],
    )(u, sc1, sh1, w1, b1, w2, b2)


def _tc_bn(v, sc2, sh2):
    def body(v_ref, sc_ref, sh_ref, o_ref):
        o_ref[...] = v_ref[...] * sc_ref[...] + sh_ref[...]

    return pl.pallas_call(
        body,
        grid=(_GRID,),
        in_specs=[pl.BlockSpec((_BM, _D), lambda i: (i, 0)),
                  pl.BlockSpec((1, _D), lambda i: (0, 0)),
                  pl.BlockSpec((1, _D), lambda i: (0, 0))],
        out_specs=pl.BlockSpec((_BM, _D), lambda i: (i, 0)),
        out_shape=jax.ShapeDtypeStruct((_N, _D), jnp.float32),
    )(v, sc2, sh2)


def kernel(h, edge_index, WQ, WK, WV, WO, bO, W1, b1, W2, b2, g1, be1,
           g2, be2):
    src = edge_index[0].astype(jnp.int32)
    dst = edge_index[1].astype(jnp.int32)

    wqkv = jnp.concatenate([WQ, WK, WV], axis=1)
    qkv = _tc_qkv(h, wqkv)
    q = qkv[:, :_D]
    k = qkv[:, _D:2 * _D]
    v = qkv[:, 2 * _D:]

    tbl = jnp.concatenate(
        [k[:, :_DHALF], k[:, _DHALF:],
         q[:, :_DHALF], q[:, _DHALF:],
         v[:, :_DHALF], v[:, _DHALF:]], axis=0)

    src2 = src.reshape(_NS, _NCHUNK, _C)
    dst2 = dst.reshape(_NS, _NCHUNK, _C)
    packed = jnp.stack([src2, dst2], axis=2).reshape(-1)

    ho = _sc_attention(tbl, packed)
    attn = jnp.concatenate([ho[:_N], ho[_NPAD:_NPAD + _N]], axis=1)

    u, s1, s2 = _tc_attn_out(attn, WO, bO.reshape(1, _D), h)
    mean1 = jnp.sum(s1[:, 0, :], axis=0) / _N
    var1 = jnp.sum(s2[:, 0, :], axis=0) / _N - mean1 * mean1
    g1i = g1 / jnp.sqrt(var1 + 1e-5)
    sc1 = g1i.reshape(1, _D)
    sh1 = (be1 - mean1 * g1i).reshape(1, _D)

    vv, t1, t2 = _tc_ffn(u, sc1, sh1, W1, b1.reshape(1, 2 * _D), W2,
                         b2.reshape(1, _D))
    mean2 = jnp.sum(t1[:, 0, :], axis=0) / _N
    var2 = jnp.sum(t2[:, 0, :], axis=0) / _N - mean2 * mean2
    g2i = g2 / jnp.sqrt(var2 + 1e-5)
    sc2 = g2i.reshape(1, _D)
    sh2 = (be2 - mean2 * g2i).reshape(1, _D)

    return _tc_bn(vv, sc2, sh2)
